# baseline (speedup 1.0000x reference)
import functools

import jax
import jax.numpy as jnp
from jax import lax
from jax.experimental import pallas as pl
from jax.experimental.pallas import tpu as pltpu
from jax.experimental.pallas import tpu_sc as plsc

TOPK_K = 128
LANES = 16
N_IMG = 64
N_CHAN = 384
SPAT = 576
ROWS2 = N_IMG * SPAT
CB = 64
NGB = CB // LANES
CBLOCKS = N_CHAN // CB
N_WORKERS = 32
TASKS_PER_W = (N_IMG * CBLOCKS) // N_WORKERS
UNROLL = 16
SP_ITERS = SPAT // UNROLL
INT_MIN32 = -(2**31)


def _sc_body(x_hbm, o_hbm, buf):
    c = lax.axis_index("c")
    s = lax.axis_index("s")
    wid = s * 2 + c

    zeros = jnp.zeros((LANES,), jnp.int32)
    ones = jnp.full((LANES,), 1, jnp.int32)
    kvec = jnp.full((LANES,), TOPK_K, jnp.int32)

    def task_body(ti, carry):
        t = wid * TASKS_PER_W + ti
        n = t // CBLOCKS
        cb = t - n * CBLOCKS
        r0 = n * SPAT
        c0 = cb * CB
        pltpu.sync_copy(x_hbm.at[pl.ds(r0, SPAT), pl.ds(c0, CB)], buf)

        def key_body(it, kc):
            sp0 = it * UNROLL
            for u in range(UNROLL):
                for g in range(NGB):
                    b = buf[sp0 + u, pl.ds(g * LANES, LANES)]
                    buf[sp0 + u, pl.ds(g * LANES, LANES)] = b ^ (
                        (b >> 31) & jnp.int32(0x7FFFFFFF)
                    )
            return kc

        lax.fori_loop(0, SP_ITERS, key_body, 0)

        def bit_body(i, tbs):
            cand_b = [tb | ones << (31 - i) for tb in tbs]
            cand = [cb_ ^ INT_MIN32 for cb_ in cand_b]

            def cnt_body(it, cnts):
                sp0 = it * UNROLL
                new = list(cnts)
                for u in range(UNROLL):
                    for g in range(NGB):
                        kj = buf[sp0 + u, pl.ds(g * LANES, LANES)]
                        new[g] = new[g] + jnp.where(
                            kj >= cand[g], ones, zeros
                        )
                return tuple(new)

            cnts = lax.fori_loop(
                0, SP_ITERS, cnt_body, (zeros,) * NGB
            )
            return tuple(
                jnp.where(cnts[g] >= kvec, cand_b[g], tbs[g])
                for g in range(NGB)
            )

        tbs = lax.fori_loop(0, 32, bit_body, (zeros,) * NGB)
        tsig = [tb ^ INT_MIN32 for tb in tbs]

        def apply_body(it, ac):
            sp0 = it * UNROLL
            for u in range(UNROLL):
                for g in range(NGB):
                    kj = buf[sp0 + u, pl.ds(g * LANES, LANES)]
                    b = kj ^ ((kj >> 31) & jnp.int32(0x7FFFFFFF))
                    keep = kj >= tsig[g]
                    buf[sp0 + u, pl.ds(g * LANES, LANES)] = jnp.where(
                        keep, b, zeros
                    )
            return ac

        lax.fori_loop(0, SP_ITERS, apply_body, 0)
        pltpu.sync_copy(buf, o_hbm.at[pl.ds(r0, SPAT), pl.ds(c0, CB)])
        return carry

    lax.fori_loop(0, TASKS_PER_W, task_body, 0)


@jax.jit
def _sc_sparsify(xt):
    mesh = plsc.VectorSubcoreMesh(core_axis_name="c", subcore_axis_name="s")
    fn = pl.kernel(
        _sc_body,
        out_type=jax.ShapeDtypeStruct((ROWS2, N_CHAN), jnp.int32),
        mesh=mesh,
        compiler_params=pltpu.CompilerParams(
            needs_layout_passes=False, use_tc_tiling_on_sc=False
        ),
        scratch_types=[pltpu.VMEM((SPAT, CB), jnp.int32)],
    )
    return fn(xt)


def kernel(x):
    n, c, h, w = x.shape
    xr = lax.bitcast_convert_type(x, jnp.int32)
    xt = jnp.transpose(xr, (0, 2, 3, 1)).reshape(n * h * w, c)
    out = _sc_sparsify(xt)
    out = jnp.transpose(out.reshape(n, h, w, c), (0, 3, 1, 2))
    return lax.bitcast_convert_type(out, jnp.float32)


# speedup vs baseline: 1.2100x; 1.2100x over previous
import functools

import jax
import jax.numpy as jnp
from jax import lax
from jax.experimental import pallas as pl
from jax.experimental.pallas import tpu as pltpu
from jax.experimental.pallas import tpu_sc as plsc

TOPK_K = 128
LANES = 16
N_IMG = 32
N_IMG_TOT = 64
N_CHAN = 384
SPAT = 576
ROWS2 = N_IMG * SPAT
CB = 64
NGB = CB // LANES
CBLOCKS = N_CHAN // CB
N_WORKERS = 32
TASKS_PER_W = (N_IMG * CBLOCKS) // N_WORKERS
UNROLL = 16
SP_ITERS = SPAT // UNROLL
INT_MIN32 = -(2**31)


def _sc_body(x_hbm, o_hbm, buf):
    c = lax.axis_index("c")
    s = lax.axis_index("s")
    wid = s * 2 + c

    zeros = jnp.zeros((LANES,), jnp.int32)
    ones = jnp.full((LANES,), 1, jnp.int32)
    kvec = jnp.full((LANES,), TOPK_K, jnp.int32)

    def task_body(ti, carry):
        t = wid * TASKS_PER_W + ti
        n = t // CBLOCKS
        cb = t - n * CBLOCKS
        r0 = n * SPAT
        c0 = cb * CB
        pltpu.sync_copy(x_hbm.at[pl.ds(r0, SPAT), pl.ds(c0, CB)], buf)

        def key_body(it, kc):
            sp0 = it * UNROLL
            for u in range(UNROLL):
                for g in range(NGB):
                    b = buf[sp0 + u, pl.ds(g * LANES, LANES)]
                    buf[sp0 + u, pl.ds(g * LANES, LANES)] = b ^ (
                        (b >> 31) & jnp.int32(0x7FFFFFFF)
                    )
            return kc

        lax.fori_loop(0, SP_ITERS, key_body, 0)

        def bit_body(i, tbs):
            cand_b = [tb | ones << (31 - i) for tb in tbs]
            cand = [cb_ ^ INT_MIN32 for cb_ in cand_b]

            def cnt_body(it, cnts):
                sp0 = it * UNROLL
                new = list(cnts)
                for u in range(UNROLL):
                    for g in range(NGB):
                        kj = buf[sp0 + u, pl.ds(g * LANES, LANES)]
                        new[g] = new[g] + jnp.where(
                            kj >= cand[g], ones, zeros
                        )
                return tuple(new)

            cnts = lax.fori_loop(
                0, SP_ITERS, cnt_body, (zeros,) * NGB
            )
            return tuple(
                jnp.where(cnts[g] >= kvec, cand_b[g], tbs[g])
                for g in range(NGB)
            )

        tbs = lax.fori_loop(0, 32, bit_body, (zeros,) * NGB)
        tsig = [tb ^ INT_MIN32 for tb in tbs]

        def apply_body(it, ac):
            sp0 = it * UNROLL
            for u in range(UNROLL):
                for g in range(NGB):
                    kj = buf[sp0 + u, pl.ds(g * LANES, LANES)]
                    b = kj ^ ((kj >> 31) & jnp.int32(0x7FFFFFFF))
                    keep = kj >= tsig[g]
                    buf[sp0 + u, pl.ds(g * LANES, LANES)] = jnp.where(
                        keep, b, zeros
                    )
            return ac

        lax.fori_loop(0, SP_ITERS, apply_body, 0)
        pltpu.sync_copy(buf, o_hbm.at[pl.ds(r0, SPAT), pl.ds(c0, CB)])
        return carry

    lax.fori_loop(0, TASKS_PER_W, task_body, 0)


@jax.jit
def _sc_sparsify(xt):
    mesh = plsc.VectorSubcoreMesh(core_axis_name="c", subcore_axis_name="s")
    fn = pl.kernel(
        _sc_body,
        out_type=jax.ShapeDtypeStruct((ROWS2, N_CHAN), jnp.int32),
        mesh=mesh,
        compiler_params=pltpu.CompilerParams(
            needs_layout_passes=False, use_tc_tiling_on_sc=False
        ),
        scratch_types=[pltpu.VMEM((SPAT, CB), jnp.int32)],
    )
    return fn(xt)


TC_BLOCK = 256
ROW_LEN = SPAT


def _tc_body(x_ref, o_ref):
    xb = x_ref[...]
    b = lax.bitcast_convert_type(xb, jnp.int32)
    ub = lax.bitcast_convert_type(xb, jnp.uint32)
    ukey = jnp.where(b < 0, ~ub, ub | jnp.uint32(0x80000000))

    def bit_step(i, t):
        bit = jnp.uint32(31) - i.astype(jnp.uint32)
        cand = t | (jnp.uint32(1) << bit)
        cnt = jnp.sum((ukey >= cand).astype(jnp.int32), axis=1, keepdims=True)
        return jnp.where(cnt >= TOPK_K, cand, t)

    t0 = jnp.zeros((xb.shape[0], 1), jnp.uint32)
    t = lax.fori_loop(0, 32, bit_step, t0)
    o_ref[...] = jnp.where(ukey >= t, xb, 0.0)


def _tc_sparsify(xr):
    rows = xr.shape[0]
    return pl.pallas_call(
        _tc_body,
        grid=(rows // TC_BLOCK,),
        in_specs=[pl.BlockSpec((TC_BLOCK, ROW_LEN), lambda i: (i, 0))],
        out_specs=pl.BlockSpec((TC_BLOCK, ROW_LEN), lambda i: (i, 0)),
        out_shape=jax.ShapeDtypeStruct((rows, ROW_LEN), xr.dtype),
    )(xr)


def kernel(x):
    n, c, h, w = x.shape
    xs = lax.bitcast_convert_type(x[:N_IMG], jnp.int32)
    xt = jnp.transpose(xs, (0, 2, 3, 1)).reshape(N_IMG * h * w, c)
    sc = _sc_sparsify(xt)
    sc_out = lax.bitcast_convert_type(
        jnp.transpose(sc.reshape(N_IMG, h, w, c), (0, 3, 1, 2)), jnp.float32
    )
    tc = _tc_sparsify(x[N_IMG:].reshape((n - N_IMG) * c, h * w))
    tc_out = tc.reshape(n - N_IMG, c, h, w)
    return jnp.concatenate([sc_out, tc_out], axis=0)


# speedup vs baseline: 1.6470x; 1.3612x over previous
import functools

import jax
import jax.numpy as jnp
from jax import lax
from jax.experimental import pallas as pl
from jax.experimental.pallas import tpu as pltpu
from jax.experimental.pallas import tpu_sc as plsc

TOPK_K = 128
LANES = 16
N_IMG = 32
N_IMG_TOT = 64
N_CHAN = 384
SPAT = 576
ROWS2 = N_IMG * SPAT
CB = 64
NGB = CB // LANES
CBLOCKS = N_CHAN // CB
N_WORKERS = 32
TASKS_PER_W = (N_IMG * CBLOCKS) // N_WORKERS
UNROLL = 16
SP_ITERS = SPAT // UNROLL
INT_MIN32 = -(2**31)


def _sc_body(x_hbm, o_hbm, buf):
    c = lax.axis_index("c")
    s = lax.axis_index("s")
    wid = s * 2 + c

    zeros = jnp.zeros((LANES,), jnp.int32)
    ones = jnp.full((LANES,), 1, jnp.int32)
    kvec = jnp.full((LANES,), TOPK_K, jnp.int32)

    def task_body(ti, carry):
        t = wid * TASKS_PER_W + ti
        n = t // CBLOCKS
        cb = t - n * CBLOCKS
        r0 = n * SPAT
        c0 = cb * CB
        pltpu.sync_copy(x_hbm.at[pl.ds(r0, SPAT), pl.ds(c0, CB)], buf)

        def key_body(it, kc):
            sp0 = it * UNROLL
            for u in range(UNROLL):
                for g in range(NGB):
                    b = buf[sp0 + u, pl.ds(g * LANES, LANES)]
                    buf[sp0 + u, pl.ds(g * LANES, LANES)] = b ^ (
                        (b >> 31) & jnp.int32(0x7FFFFFFF)
                    )
            return kc

        lax.fori_loop(0, SP_ITERS, key_body, 0)

        def bit_body(i, tbs):
            cand_b = [tb | ones << (31 - i) for tb in tbs]
            cand = [cb_ ^ INT_MIN32 for cb_ in cand_b]

            def cnt_body(it, cnts):
                sp0 = it * UNROLL
                new = list(cnts)
                for u in range(UNROLL):
                    for g in range(NGB):
                        kj = buf[sp0 + u, pl.ds(g * LANES, LANES)]
                        new[g] = new[g] + jnp.where(
                            kj >= cand[g], ones, zeros
                        )
                return tuple(new)

            cnts = lax.fori_loop(
                0, SP_ITERS, cnt_body, (zeros,) * NGB
            )
            return tuple(
                jnp.where(cnts[g] >= kvec, cand_b[g], tbs[g])
                for g in range(NGB)
            )

        tbs = lax.fori_loop(0, 32, bit_body, (zeros,) * NGB)
        tsig = [tb ^ INT_MIN32 for tb in tbs]

        def apply_body(it, ac):
            sp0 = it * UNROLL
            for u in range(UNROLL):
                for g in range(NGB):
                    kj = buf[sp0 + u, pl.ds(g * LANES, LANES)]
                    b = kj ^ ((kj >> 31) & jnp.int32(0x7FFFFFFF))
                    keep = kj >= tsig[g]
                    buf[sp0 + u, pl.ds(g * LANES, LANES)] = jnp.where(
                        keep, b, zeros
                    )
            return ac

        lax.fori_loop(0, SP_ITERS, apply_body, 0)
        pltpu.sync_copy(buf, o_hbm.at[pl.ds(r0, SPAT), pl.ds(c0, CB)])
        return carry

    lax.fori_loop(0, TASKS_PER_W, task_body, 0)


@jax.jit
def _sc_sparsify(xt):
    mesh = plsc.VectorSubcoreMesh(core_axis_name="c", subcore_axis_name="s")
    fn = pl.kernel(
        _sc_body,
        out_type=jax.ShapeDtypeStruct((ROWS2, N_CHAN), jnp.int32),
        mesh=mesh,
        compiler_params=pltpu.CompilerParams(
            needs_layout_passes=False, use_tc_tiling_on_sc=False
        ),
        scratch_types=[pltpu.VMEM((SPAT, CB), jnp.int32)],
    )
    return fn(xt)


def _tc_body(x_ref, o_ref):
    b = x_ref[...]
    ub = lax.bitcast_convert_type(b, jnp.uint32)
    ukey = jnp.where(b < 0, ~ub, ub | jnp.uint32(0x80000000))

    def bit_step(i, t):
        bit = jnp.uint32(31) - i.astype(jnp.uint32)
        cand = t | (jnp.uint32(1) << bit)
        cnt = jnp.sum((ukey >= cand).astype(jnp.int32), axis=0, keepdims=True)
        return jnp.where(cnt >= TOPK_K, cand, t)

    t0 = jnp.zeros((1, N_CHAN), jnp.uint32)
    t = lax.fori_loop(0, 32, bit_step, t0)
    o_ref[...] = jnp.where(ukey >= t, b, 0)


def _tc_sparsify(xt):
    rows = xt.shape[0]
    return pl.pallas_call(
        _tc_body,
        grid=(rows // SPAT,),
        in_specs=[pl.BlockSpec((SPAT, N_CHAN), lambda i: (i, 0))],
        out_specs=pl.BlockSpec((SPAT, N_CHAN), lambda i: (i, 0)),
        out_shape=jax.ShapeDtypeStruct((rows, N_CHAN), xt.dtype),
    )(xt)


def kernel(x):
    n, c, h, w = x.shape
    xi = lax.bitcast_convert_type(x, jnp.int32)
    xt = jnp.transpose(xi, (0, 2, 3, 1)).reshape(n * h * w, c)
    sc = _sc_sparsify(xt[: N_IMG * SPAT])
    tc = _tc_sparsify(xt[N_IMG * SPAT :])
    ot = jnp.concatenate([sc, tc], axis=0).reshape(n, h, w, c)
    out = jnp.transpose(ot, (0, 3, 1, 2))
    return lax.bitcast_convert_type(out, jnp.float32)
